# Initial kernel scaffold; baseline (speedup 1.0000x reference)
import jax
import jax.numpy as jnp
from jax.experimental import pallas as pl


def kernel(x, edge_index, W1, b1, W2, b2, W3, b3, Wz, Uz, bz, Wr, Ur, br, Wh, Uh, bh):
    raise NotImplementedError("write your pallas kernel here")


# speedup vs baseline: 7.5889x; 7.5889x over previous
import functools

import jax
import jax.numpy as jnp
from jax import lax
from jax.experimental import pallas as pl
from jax.experimental.pallas import tpu as pltpu
from jax.experimental.pallas import tpu_sc as plsc

N = 10000
E = 320000
D = 128
M = 128

NC = 2
NS = 16
NW = NC * NS
TILE_E = E // NW
CHUNK = 80
NCHUNK = TILE_E // CHUNK
N_PAD = 10240
ROWS_PER_TILE = N_PAD // NS

BN = 2000
GRID = N // BN


def _mlp_body(x_ref, w1_ref, b1_ref, w2_ref, b2_ref, w3_ref, b3_ref, y_ref):
    h = jnp.dot(x_ref[...], w1_ref[...], preferred_element_type=jnp.float32)
    h = jnp.maximum(h + b1_ref[...], 0.0)
    h = jnp.dot(h, w2_ref[...], preferred_element_type=jnp.float32)
    h = jnp.maximum(h + b2_ref[...], 0.0)
    y_ref[...] = (
        jnp.dot(h, w3_ref[...], preferred_element_type=jnp.float32) + b3_ref[...]
    )


def _mlp(x, W1, b1, W2, b2, W3, b3):
    row_spec = pl.BlockSpec((BN, D), lambda i: (i, 0))
    w_spec = pl.BlockSpec((D, M), lambda i: (0, 0))
    b_spec = pl.BlockSpec((1, M), lambda i: (0, 0))
    return pl.pallas_call(
        _mlp_body,
        grid=(GRID,),
        in_specs=[row_spec, w_spec, b_spec,
                  pl.BlockSpec((M, M), lambda i: (0, 0)), b_spec,
                  pl.BlockSpec((M, D), lambda i: (0, 0)),
                  pl.BlockSpec((1, D), lambda i: (0, 0))],
        out_specs=pl.BlockSpec((BN, D), lambda i: (i, 0)),
        out_shape=jax.ShapeDtypeStruct((N, D), jnp.float32),
    )(x, W1, b1.reshape(1, M), W2, b2.reshape(1, M), W3, b3.reshape(1, D))


def _edge_aggregate(y, src, dst, zeros):
    mesh = plsc.VectorSubcoreMesh(core_axis_name="c", subcore_axis_name="s")

    @functools.partial(
        pl.kernel,
        mesh=mesh,
        out_type=jax.ShapeDtypeStruct((NC, N_PAD, D), jnp.float32),
        scratch_types=[
            pltpu.VMEM((NCHUNK, CHUNK), jnp.int32),
            pltpu.VMEM((NCHUNK, CHUNK), jnp.int32),
            pltpu.VMEM((CHUNK, D), jnp.float32),
            pltpu.VMEM_SHARED((N_PAD, D), jnp.float32),
            pltpu.SemaphoreType.DMA,
        ],
    )
    def k(y_hbm, src_hbm, dst_hbm, zeros_hbm, out_hbm,
          src_v, dst_v, rows_v, agg_sh, sem):
        c = lax.axis_index("c")
        s = lax.axis_index("s")
        wid = c * NS + s
        pltpu.sync_copy(zeros_hbm,
                        agg_sh.at[pl.ds(s * ROWS_PER_TILE, ROWS_PER_TILE)])
        pltpu.sync_copy(src_hbm.at[wid], src_v)
        pltpu.sync_copy(dst_hbm.at[wid], dst_v)
        plsc.subcore_barrier()

        def body(j, carry):
            pltpu.async_copy(y_hbm.at[src_v.at[j]], rows_v, sem).wait()
            pltpu.sync_copy(rows_v, agg_sh.at[dst_v.at[j]], add=True)
            return carry

        lax.fori_loop(0, NCHUNK, body, 0)
        plsc.subcore_barrier()
        pltpu.sync_copy(agg_sh.at[pl.ds(s * ROWS_PER_TILE, ROWS_PER_TILE)],
                        out_hbm.at[c, pl.ds(s * ROWS_PER_TILE, ROWS_PER_TILE)])

    return k(y, src, dst, zeros)


def _gru_body(p0_ref, p1_ref, x_ref,
              wz_ref, uz_ref, bz_ref, wr_ref, ur_ref, br_ref,
              wh_ref, uh_ref, bh_ref, out_ref):
    agg = p0_ref[...] + p1_ref[...]
    xv = x_ref[...]
    dot = lambda a, b: jnp.dot(a, b[...], preferred_element_type=jnp.float32)
    z = jax.nn.sigmoid(dot(agg, wz_ref) + dot(xv, uz_ref) + bz_ref[...])
    r = jax.nn.sigmoid(dot(agg, wr_ref) + dot(xv, ur_ref) + br_ref[...])
    n = jnp.tanh(dot(agg, wh_ref) + r * dot(xv, uh_ref) + bh_ref[...])
    out_ref[...] = (1.0 - z) * n + z * xv


def _gru(p0, p1, x, Wz, Uz, bz, Wr, Ur, br, Wh, Uh, bh):
    row_spec = pl.BlockSpec((BN, D), lambda i: (i, 0))
    w_spec = pl.BlockSpec((D, D), lambda i: (0, 0))
    b_spec = pl.BlockSpec((1, D), lambda i: (0, 0))
    return pl.pallas_call(
        _gru_body,
        grid=(GRID,),
        in_specs=[row_spec, row_spec, row_spec,
                  w_spec, w_spec, b_spec,
                  w_spec, w_spec, b_spec,
                  w_spec, w_spec, b_spec],
        out_specs=row_spec,
        out_shape=jax.ShapeDtypeStruct((N, D), jnp.float32),
    )(p0, p1, x, Wz, Uz, bz.reshape(1, D), Wr, Ur, br.reshape(1, D),
      Wh, Uh, bh.reshape(1, D))


def kernel(x, edge_index, W1, b1, W2, b2, W3, b3,
           Wz, Uz, bz, Wr, Ur, br, Wh, Uh, bh):
    src = edge_index[0].reshape(NW, NCHUNK, CHUNK)
    dst = edge_index[1].reshape(NW, NCHUNK, CHUNK)
    zeros = jnp.zeros((ROWS_PER_TILE, D), jnp.float32)
    y = _mlp(x, W1, b1, W2, b2, W3, b3)
    partials = _edge_aggregate(y, src, dst, zeros)
    return _gru(partials[0, :N], partials[1, :N], x,
                Wz, Uz, bz, Wr, Ur, br, Wh, Uh, bh)
